# Initial kernel scaffold; baseline (speedup 1.0000x reference)
import jax
import jax.numpy as jnp
from jax.experimental import pallas as pl


def kernel(x_p, edge_indices, edge_feats, W_edge, b_edge, W_mlp, b_mlp):
    raise NotImplementedError("write your pallas kernel here")


# speedup vs baseline: 1.0477x; 1.0477x over previous
import functools

import jax
import jax.numpy as jnp
from jax.experimental import pallas as pl
from jax.experimental.pallas import tpu as pltpu

N_NODES = 10000
WIDTH = 128
E2 = 640000

EDGE_BLK = 1024
NODE_BLK = 1000


def _edge_mlp_body(src_ref, dst_ref, ef_ref, w1_ref, w2_ref, b_ref, out_ref):
    src = src_ref[...]
    dst = dst_ref[...]
    diffs = dst - src
    h = jnp.dot(diffs, w1_ref[...], preferred_element_type=jnp.float32)
    ef = ef_ref[...]
    h = h + ef[:, 0:1] * w2_ref[0:1, :] + ef[:, 1:2] * w2_ref[1:2, :]
    h = jnp.maximum(h + b_ref[...], 0.0)
    out_ref[...] = diffs + h


def _edge_mlp(src_rows, dst_rows, ef2, W1, W2, b_edge):
    nblk = E2 // EDGE_BLK
    return pl.pallas_call(
        _edge_mlp_body,
        grid=(nblk,),
        in_specs=[
            pl.BlockSpec((EDGE_BLK, WIDTH), lambda i: (i, 0)),
            pl.BlockSpec((EDGE_BLK, WIDTH), lambda i: (i, 0)),
            pl.BlockSpec((EDGE_BLK, 2), lambda i: (i, 0)),
            pl.BlockSpec((WIDTH, WIDTH), lambda i: (0, 0)),
            pl.BlockSpec((2, WIDTH), lambda i: (0, 0)),
            pl.BlockSpec((1, WIDTH), lambda i: (0, 0)),
        ],
        out_specs=pl.BlockSpec((EDGE_BLK, WIDTH), lambda i: (i, 0)),
        out_shape=jax.ShapeDtypeStruct((E2, WIDTH), jnp.float32),
    )(src_rows, dst_rows, ef2, W1, W2, b_edge)


def _node_mlp_body(x_ref, mx_ref, wm1_ref, wm2_ref, b_ref, out_ref):
    x = x_ref[...]
    mx = mx_ref[...]
    mx = jnp.where(jnp.isneginf(mx), 0.0, mx)
    h = jnp.dot(x, wm1_ref[...], preferred_element_type=jnp.float32)
    h = h + jnp.dot(mx, wm2_ref[...], preferred_element_type=jnp.float32)
    h = jnp.maximum(h + b_ref[...], 0.0)
    out_ref[...] = x + h


def _node_mlp(x_p, maxes, Wm1, Wm2, b_mlp):
    nblk = N_NODES // NODE_BLK
    return pl.pallas_call(
        _node_mlp_body,
        grid=(nblk,),
        in_specs=[
            pl.BlockSpec((NODE_BLK, WIDTH), lambda i: (i, 0)),
            pl.BlockSpec((NODE_BLK, WIDTH), lambda i: (i, 0)),
            pl.BlockSpec((WIDTH, WIDTH), lambda i: (0, 0)),
            pl.BlockSpec((WIDTH, WIDTH), lambda i: (0, 0)),
            pl.BlockSpec((1, WIDTH), lambda i: (0, 0)),
        ],
        out_specs=pl.BlockSpec((NODE_BLK, WIDTH), lambda i: (i, 0)),
        out_shape=jax.ShapeDtypeStruct((N_NODES, WIDTH), jnp.float32),
    )(x_p, maxes, Wm1, Wm2, b_mlp)


def kernel(x_p, edge_indices, edge_feats, W_edge, b_edge, W_mlp, b_mlp):
    e0 = jnp.concatenate([edge_indices[0], edge_indices[1]])
    e1 = jnp.concatenate([edge_indices[1], edge_indices[0]])
    ef2 = jnp.concatenate([edge_feats, edge_feats], axis=0)
    W1 = W_edge[:WIDTH]
    W2 = W_edge[WIDTH:]
    Wm1 = W_mlp[:WIDTH]
    Wm2 = W_mlp[WIDTH:]

    src_rows = jnp.take(x_p, e0, axis=0)
    dst_rows = jnp.take(x_p, e1, axis=0)

    e_msg = _edge_mlp(src_rows, dst_rows, ef2, W1, W2,
                      b_edge.reshape(1, WIDTH))

    maxes = jax.ops.segment_max(e_msg, e1, num_segments=N_NODES)

    return _node_mlp(x_p, maxes, Wm1, Wm2, b_mlp.reshape(1, WIDTH))


# speedup vs baseline: 2.6939x; 2.5713x over previous
import functools

import jax
import jax.numpy as jnp
from jax import lax
from jax.experimental import pallas as pl
from jax.experimental.pallas import tpu as pltpu
from jax.experimental.pallas import tpu_sc as plsc

N_NODES = 10000
WIDTH = 128
E2 = 640000

EDGE_BLK = 1024
NODE_BLK = 1000

NWORKERS = 32
GATHER_CHUNK = 400
EDGES_PER_W = E2 // NWORKERS


def _gather_body(x_hbm, e0_hbm, e1_hbm, src_hbm, dst_hbm,
                 idx0_v, idx1_v, src_v, dst_v, sem0, sem1):
    wid = lax.axis_index("s") * 2 + lax.axis_index("c")
    wbase = wid * EDGES_PER_W

    def chunk(c, _):
        base = wbase + c * GATHER_CHUNK
        pltpu.sync_copy(e0_hbm.at[pl.ds(base, GATHER_CHUNK)], idx0_v)
        pltpu.sync_copy(e1_hbm.at[pl.ds(base, GATHER_CHUNK)], idx1_v)
        cp0 = pltpu.async_copy(x_hbm.at[idx0_v], src_v, sem0)
        cp1 = pltpu.async_copy(x_hbm.at[idx1_v], dst_v, sem1)
        cp0.wait()
        cp1.wait()
        pltpu.sync_copy(src_v, src_hbm.at[pl.ds(base, GATHER_CHUNK), :])
        pltpu.sync_copy(dst_v, dst_hbm.at[pl.ds(base, GATHER_CHUNK), :])
        return ()

    lax.fori_loop(0, EDGES_PER_W // GATHER_CHUNK, chunk, ())


def _sc_gather(x_p, e0, e1):
    mesh = plsc.VectorSubcoreMesh(core_axis_name="c", subcore_axis_name="s")
    f = pl.kernel(
        _gather_body,
        out_type=[
            jax.ShapeDtypeStruct((E2, WIDTH), jnp.float32),
            jax.ShapeDtypeStruct((E2, WIDTH), jnp.float32),
        ],
        mesh=mesh,
        scratch_types=[
            pltpu.VMEM((GATHER_CHUNK,), jnp.int32),
            pltpu.VMEM((GATHER_CHUNK,), jnp.int32),
            pltpu.VMEM((GATHER_CHUNK, WIDTH), jnp.float32),
            pltpu.VMEM((GATHER_CHUNK, WIDTH), jnp.float32),
            pltpu.SemaphoreType.DMA,
            pltpu.SemaphoreType.DMA,
        ],
    )
    return f(x_p, e0, e1)


def _edge_mlp_body(src_ref, dst_ref, ef_ref, w1_ref, w2_ref, b_ref, out_ref):
    src = src_ref[...]
    dst = dst_ref[...]
    diffs = dst - src
    h = jnp.dot(diffs, w1_ref[...], preferred_element_type=jnp.float32)
    ef = ef_ref[...]
    h = h + ef[:, 0:1] * w2_ref[0:1, :] + ef[:, 1:2] * w2_ref[1:2, :]
    h = jnp.maximum(h + b_ref[...], 0.0)
    out_ref[...] = diffs + h


def _edge_mlp(src_rows, dst_rows, ef2, W1, W2, b_edge):
    nblk = E2 // EDGE_BLK
    return pl.pallas_call(
        _edge_mlp_body,
        grid=(nblk,),
        in_specs=[
            pl.BlockSpec((EDGE_BLK, WIDTH), lambda i: (i, 0)),
            pl.BlockSpec((EDGE_BLK, WIDTH), lambda i: (i, 0)),
            pl.BlockSpec((EDGE_BLK, 2), lambda i: (i, 0)),
            pl.BlockSpec((WIDTH, WIDTH), lambda i: (0, 0)),
            pl.BlockSpec((2, WIDTH), lambda i: (0, 0)),
            pl.BlockSpec((1, WIDTH), lambda i: (0, 0)),
        ],
        out_specs=pl.BlockSpec((EDGE_BLK, WIDTH), lambda i: (i, 0)),
        out_shape=jax.ShapeDtypeStruct((E2, WIDTH), jnp.float32),
    )(src_rows, dst_rows, ef2, W1, W2, b_edge)


def _node_mlp_body(x_ref, mx_ref, wm1_ref, wm2_ref, b_ref, out_ref):
    x = x_ref[...]
    mx = mx_ref[...]
    mx = jnp.where(jnp.isneginf(mx), 0.0, mx)
    h = jnp.dot(x, wm1_ref[...], preferred_element_type=jnp.float32)
    h = h + jnp.dot(mx, wm2_ref[...], preferred_element_type=jnp.float32)
    h = jnp.maximum(h + b_ref[...], 0.0)
    out_ref[...] = x + h


def _node_mlp(x_p, maxes, Wm1, Wm2, b_mlp):
    nblk = N_NODES // NODE_BLK
    return pl.pallas_call(
        _node_mlp_body,
        grid=(nblk,),
        in_specs=[
            pl.BlockSpec((NODE_BLK, WIDTH), lambda i: (i, 0)),
            pl.BlockSpec((NODE_BLK, WIDTH), lambda i: (i, 0)),
            pl.BlockSpec((WIDTH, WIDTH), lambda i: (0, 0)),
            pl.BlockSpec((WIDTH, WIDTH), lambda i: (0, 0)),
            pl.BlockSpec((1, WIDTH), lambda i: (0, 0)),
        ],
        out_specs=pl.BlockSpec((NODE_BLK, WIDTH), lambda i: (i, 0)),
        out_shape=jax.ShapeDtypeStruct((N_NODES, WIDTH), jnp.float32),
    )(x_p, maxes, Wm1, Wm2, b_mlp)


def kernel(x_p, edge_indices, edge_feats, W_edge, b_edge, W_mlp, b_mlp):
    e0 = jnp.concatenate([edge_indices[0], edge_indices[1]])
    e1 = jnp.concatenate([edge_indices[1], edge_indices[0]])
    ef2 = jnp.concatenate([edge_feats, edge_feats], axis=0)
    W1 = W_edge[:WIDTH]
    W2 = W_edge[WIDTH:]
    Wm1 = W_mlp[:WIDTH]
    Wm2 = W_mlp[WIDTH:]

    src_rows, dst_rows = _sc_gather(x_p, e0, e1)

    e_msg = _edge_mlp(src_rows, dst_rows, ef2, W1, W2,
                      b_edge.reshape(1, WIDTH))

    maxes = jax.ops.segment_max(e_msg, e1, num_segments=N_NODES)

    return _node_mlp(x_p, maxes, Wm1, Wm2, b_mlp.reshape(1, WIDTH))


# speedup vs baseline: 2.7497x; 1.0207x over previous
import functools

import jax
import jax.numpy as jnp
from jax import lax
from jax.experimental import pallas as pl
from jax.experimental.pallas import tpu as pltpu
from jax.experimental.pallas import tpu_sc as plsc

N_NODES = 10000
WIDTH = 128
E2 = 640000

EDGE_BLK = 1024
NODE_BLK = 1000

NWORKERS = 32
GATHER_CHUNK = 400
EDGES_PER_W = E2 // NWORKERS


def _gather_body(x_hbm, e0_hbm, e1_hbm, src_hbm, dst_hbm,
                 idx0_v, idx1_v, src_v, dst_v, sem0, sem1):
    wid = lax.axis_index("s") * 2 + lax.axis_index("c")
    wbase = wid * EDGES_PER_W

    def chunk(c, _):
        base = wbase + c * GATHER_CHUNK
        pltpu.sync_copy(e0_hbm.at[pl.ds(base, GATHER_CHUNK)], idx0_v)
        pltpu.sync_copy(e1_hbm.at[pl.ds(base, GATHER_CHUNK)], idx1_v)
        cp0 = pltpu.async_copy(x_hbm.at[idx0_v], src_v, sem0)
        cp1 = pltpu.async_copy(x_hbm.at[idx1_v], dst_v, sem1)
        cp0.wait()
        cp1.wait()
        pltpu.sync_copy(src_v, src_hbm.at[pl.ds(base, GATHER_CHUNK), :])
        pltpu.sync_copy(dst_v, dst_hbm.at[pl.ds(base, GATHER_CHUNK), :])
        return ()

    lax.fori_loop(0, EDGES_PER_W // GATHER_CHUNK, chunk, ())


def _sc_gather(x_p, e0, e1):
    mesh = plsc.VectorSubcoreMesh(core_axis_name="c", subcore_axis_name="s")
    f = pl.kernel(
        _gather_body,
        out_type=[
            jax.ShapeDtypeStruct((E2, WIDTH), jnp.float32),
            jax.ShapeDtypeStruct((E2, WIDTH), jnp.float32),
        ],
        mesh=mesh,
        scratch_types=[
            pltpu.VMEM((GATHER_CHUNK,), jnp.int32),
            pltpu.VMEM((GATHER_CHUNK,), jnp.int32),
            pltpu.VMEM((GATHER_CHUNK, WIDTH), jnp.float32),
            pltpu.VMEM((GATHER_CHUNK, WIDTH), jnp.float32),
            pltpu.SemaphoreType.DMA,
            pltpu.SemaphoreType.DMA,
        ],
    )
    return f(x_p, e0, e1)


NPW = 313
ACC_ROWS = NPW + 1
W_SCAN = 5120
RING = 8192
FLUSH = 256


def _scatter_max_body(e1_hbm, emsg_hbm, maxes_hbm, hld_hbm, hid_hbm,
                      dwin, ring_ld, ring_id, lds_buf, ids_buf, rows_v,
                      acc, sem):
    wid = lax.axis_index("s") * 2 + lax.axis_index("c")
    lo = wid * NPW
    hi = lo + NPW
    hbase = wid * E2
    lane = lax.iota(jnp.int32, 16)
    neg_inf = jnp.full((16,), -jnp.inf, dtype=jnp.float32)

    def init(i, _):
        acc[pl.ds(i * 16, 16)] = neg_inf
        return ()

    lax.fori_loop(0, ACC_ROWS * WIDTH // 16, init, ())

    def flush_cond_maker(goal):
        def cond(carry):
            return carry[0] + goal <= carry[1]
        return cond

    def flush_body(carry):
        fl, p = carry
        flr = pl.multiple_of(fl % RING, FLUSH)
        flh = pl.multiple_of(hbase + fl, FLUSH)
        pltpu.sync_copy(ring_ld.at[pl.ds(flr, FLUSH)],
                        hld_hbm.at[pl.ds(flh, FLUSH)])
        pltpu.sync_copy(ring_id.at[pl.ds(flr, FLUSH)],
                        hid_hbm.at[pl.ds(flh, FLUSH)])
        return (fl + FLUSH, p)

    def window(w, carry):
        p0, fl0 = carry
        pltpu.sync_copy(e1_hbm.at[pl.ds(w * W_SCAN, W_SCAN)], dwin)

        def scan_grp(g, p):
            d = dwin[pl.ds(g * 16, 16)]
            m = (d >= lo) & (d < hi)
            c = plsc.cumsum(jnp.where(m, 1, 0).astype(jnp.int32))
            pos = (p + c - 1) % RING
            plsc.store_scatter(ring_ld, [pos], d - lo, mask=m)
            ids = (w * W_SCAN + g * 16) + lane
            plsc.store_scatter(ring_id, [pos], ids, mask=m)
            return p + jnp.max(c)

        p1 = lax.fori_loop(0, W_SCAN // 16, scan_grp, p0)
        fl1, _ = lax.while_loop(flush_cond_maker(FLUSH), flush_body, (fl0, p1))
        return (p1, fl1)

    p, fl = lax.fori_loop(0, E2 // W_SCAN, window, (0, 0))

    n_pad = (-(p - fl)) % FLUSH

    def pad_grp(t, _):
        pos = p + t * 16 + lane
        m = pos < p + n_pad
        plsc.store_scatter(ring_ld, [pos % RING],
                           jnp.full((16,), NPW, jnp.int32), mask=m)
        plsc.store_scatter(ring_id, [pos % RING],
                           jnp.zeros((16,), jnp.int32), mask=m)
        return ()

    lax.fori_loop(0, FLUSH // 16, pad_grp, ())
    fl, _ = lax.while_loop(flush_cond_maker(1), flush_body, (fl, p + n_pad))

    def batch(b, _):
        bofs = pl.multiple_of(hbase + b * FLUSH, FLUSH)
        pltpu.sync_copy(hld_hbm.at[pl.ds(bofs, FLUSH)], lds_buf)
        pltpu.sync_copy(hid_hbm.at[pl.ds(bofs, FLUSH)], ids_buf)
        pltpu.async_copy(emsg_hbm.at[ids_buf], rows_v, sem).wait()

        def grp(g, _):
            lds16 = lds_buf[pl.ds(g * 16, 16)]
            for j in range(16):
                dj = jnp.sum(jnp.where(lane == j, lds16, 0))
                abase = pl.multiple_of(dj * WIDTH, WIDTH)
                for f in range(8):
                    sl = pl.ds(abase + f * 16, 16)
                    acc[sl] = jnp.maximum(
                        acc[sl], rows_v[g * 16 + j, pl.ds(f * 16, 16)])
            return ()

        lax.fori_loop(0, FLUSH // 16, grp, ())
        return ()

    lax.fori_loop(0, fl // FLUSH, batch, ())

    pltpu.sync_copy(acc.at[pl.ds(0, NPW * WIDTH)],
                    maxes_hbm.at[pl.ds(pl.multiple_of(wid * NPW * WIDTH, 64),
                                       NPW * WIDTH)])


def _sc_scatter_max(e1, e_msg):
    mesh = plsc.VectorSubcoreMesh(core_axis_name="c", subcore_axis_name="s")
    f = pl.kernel(
        _scatter_max_body,
        out_type=[
            jax.ShapeDtypeStruct((NWORKERS * NPW * WIDTH,), jnp.float32),
            jax.ShapeDtypeStruct((NWORKERS * E2,), jnp.int32),
            jax.ShapeDtypeStruct((NWORKERS * E2,), jnp.int32),
        ],
        mesh=mesh,
        compiler_params=pltpu.CompilerParams(needs_layout_passes=False),
        scratch_types=[
            pltpu.VMEM((W_SCAN,), jnp.int32),
            pltpu.VMEM((RING,), jnp.int32),
            pltpu.VMEM((RING,), jnp.int32),
            pltpu.VMEM((FLUSH,), jnp.int32),
            pltpu.VMEM((FLUSH,), jnp.int32),
            pltpu.VMEM((FLUSH, WIDTH), jnp.float32),
            pltpu.VMEM((ACC_ROWS * WIDTH,), jnp.float32),
            pltpu.SemaphoreType.DMA,
        ],
    )
    maxes_flat, _, _ = f(e1, e_msg)
    return maxes_flat.reshape(NWORKERS * NPW, WIDTH)[:N_NODES]


def _edge_mlp_body(src_ref, dst_ref, ef_ref, w1_ref, w2_ref, b_ref, out_ref):
    src = src_ref[...]
    dst = dst_ref[...]
    diffs = dst - src
    h = jnp.dot(diffs, w1_ref[...], preferred_element_type=jnp.float32)
    ef = ef_ref[...]
    h = h + ef[:, 0:1] * w2_ref[0:1, :] + ef[:, 1:2] * w2_ref[1:2, :]
    h = jnp.maximum(h + b_ref[...], 0.0)
    out_ref[...] = diffs + h


def _edge_mlp(src_rows, dst_rows, ef2, W1, W2, b_edge):
    nblk = E2 // EDGE_BLK
    return pl.pallas_call(
        _edge_mlp_body,
        grid=(nblk,),
        in_specs=[
            pl.BlockSpec((EDGE_BLK, WIDTH), lambda i: (i, 0)),
            pl.BlockSpec((EDGE_BLK, WIDTH), lambda i: (i, 0)),
            pl.BlockSpec((EDGE_BLK, 2), lambda i: (i, 0)),
            pl.BlockSpec((WIDTH, WIDTH), lambda i: (0, 0)),
            pl.BlockSpec((2, WIDTH), lambda i: (0, 0)),
            pl.BlockSpec((1, WIDTH), lambda i: (0, 0)),
        ],
        out_specs=pl.BlockSpec((EDGE_BLK, WIDTH), lambda i: (i, 0)),
        out_shape=jax.ShapeDtypeStruct((E2, WIDTH), jnp.float32),
    )(src_rows, dst_rows, ef2, W1, W2, b_edge)


def _node_mlp_body(x_ref, mx_ref, wm1_ref, wm2_ref, b_ref, out_ref):
    x = x_ref[...]
    mx = mx_ref[...]
    mx = jnp.where(jnp.isneginf(mx), 0.0, mx)
    h = jnp.dot(x, wm1_ref[...], preferred_element_type=jnp.float32)
    h = h + jnp.dot(mx, wm2_ref[...], preferred_element_type=jnp.float32)
    h = jnp.maximum(h + b_ref[...], 0.0)
    out_ref[...] = x + h


def _node_mlp(x_p, maxes, Wm1, Wm2, b_mlp):
    nblk = N_NODES // NODE_BLK
    return pl.pallas_call(
        _node_mlp_body,
        grid=(nblk,),
        in_specs=[
            pl.BlockSpec((NODE_BLK, WIDTH), lambda i: (i, 0)),
            pl.BlockSpec((NODE_BLK, WIDTH), lambda i: (i, 0)),
            pl.BlockSpec((WIDTH, WIDTH), lambda i: (0, 0)),
            pl.BlockSpec((WIDTH, WIDTH), lambda i: (0, 0)),
            pl.BlockSpec((1, WIDTH), lambda i: (0, 0)),
        ],
        out_specs=pl.BlockSpec((NODE_BLK, WIDTH), lambda i: (i, 0)),
        out_shape=jax.ShapeDtypeStruct((N_NODES, WIDTH), jnp.float32),
    )(x_p, maxes, Wm1, Wm2, b_mlp)


def kernel(x_p, edge_indices, edge_feats, W_edge, b_edge, W_mlp, b_mlp):
    e0 = jnp.concatenate([edge_indices[0], edge_indices[1]])
    e1 = jnp.concatenate([edge_indices[1], edge_indices[0]])
    ef2 = jnp.concatenate([edge_feats, edge_feats], axis=0)
    W1 = W_edge[:WIDTH]
    W2 = W_edge[WIDTH:]
    Wm1 = W_mlp[:WIDTH]
    Wm2 = W_mlp[WIDTH:]

    src_rows, dst_rows = _sc_gather(x_p, e0, e1)

    e_msg = _edge_mlp(src_rows, dst_rows, ef2, W1, W2,
                      b_edge.reshape(1, WIDTH))

    maxes = _sc_scatter_max(e1, e_msg)

    return _node_mlp(x_p, maxes, Wm1, Wm2, b_mlp.reshape(1, WIDTH))


# speedup vs baseline: 2.9303x; 1.0657x over previous
import functools

import jax
import jax.numpy as jnp
from jax import lax
from jax.experimental import pallas as pl
from jax.experimental.pallas import tpu as pltpu
from jax.experimental.pallas import tpu_sc as plsc

N_NODES = 10000
WIDTH = 128
E2 = 640000

EDGE_BLK = 1024
NODE_BLK = 1000

NWORKERS = 32
GATHER_CHUNK = 200
EDGES_PER_W = E2 // NWORKERS


def _gather_body(x_hbm, e0_hbm, e1_hbm, src_hbm, dst_hbm,
                 i0a, i1a, i0b, i1b, sva, dva, svb, dvb,
                 s0a, s1a, s0b, s1b):
    wid = lax.axis_index("s") * 2 + lax.axis_index("c")
    wbase = wid * EDGES_PER_W

    def fire(c, i0, i1, sv, dv, s0, s1):
        base = wbase + c * GATHER_CHUNK
        pltpu.sync_copy(e0_hbm.at[pl.ds(base, GATHER_CHUNK)], i0)
        pltpu.sync_copy(e1_hbm.at[pl.ds(base, GATHER_CHUNK)], i1)
        pltpu.async_copy(x_hbm.at[i0], sv, s0)
        pltpu.async_copy(x_hbm.at[i1], dv, s1)

    def drain_wb(c, i0, i1, sv, dv, s0, s1):
        base = wbase + c * GATHER_CHUNK
        pltpu.make_async_copy(x_hbm.at[i0], sv, s0).wait()
        pltpu.make_async_copy(x_hbm.at[i1], dv, s1).wait()
        pltpu.sync_copy(sv, src_hbm.at[pl.ds(base, GATHER_CHUNK), :])
        pltpu.sync_copy(dv, dst_hbm.at[pl.ds(base, GATHER_CHUNK), :])

    fire(0, i0a, i1a, sva, dva, s0a, s1a)
    nch = EDGES_PER_W // GATHER_CHUNK

    def pair(cc, _):
        e = cc * 2
        fire(e + 1, i0b, i1b, svb, dvb, s0b, s1b)
        drain_wb(e, i0a, i1a, sva, dva, s0a, s1a)

        @pl.when(e + 2 < nch)
        def _():
            fire(e + 2, i0a, i1a, sva, dva, s0a, s1a)

        drain_wb(e + 1, i0b, i1b, svb, dvb, s0b, s1b)
        return ()

    lax.fori_loop(0, nch // 2, pair, ())


def _sc_gather(x_p, e0, e1):
    mesh = plsc.VectorSubcoreMesh(core_axis_name="c", subcore_axis_name="s")
    f = pl.kernel(
        _gather_body,
        out_type=[
            jax.ShapeDtypeStruct((E2, WIDTH), jnp.float32),
            jax.ShapeDtypeStruct((E2, WIDTH), jnp.float32),
        ],
        mesh=mesh,
        compiler_params=pltpu.CompilerParams(needs_layout_passes=False),
        scratch_types=(
            [pltpu.VMEM((GATHER_CHUNK,), jnp.int32)] * 4
            + [pltpu.VMEM((GATHER_CHUNK, WIDTH), jnp.float32)] * 4
            + [pltpu.SemaphoreType.DMA] * 4
        ),
    )
    return f(x_p, e0, e1)


NPW = 313
ACC_ROWS = NPW + 1
W_SCAN = 2560
RING = 4096
FLUSH = 256
NWIN = E2 // W_SCAN


def _scatter_max_body(e1_hbm, emsg_hbm, maxes_hbm, hld_hbm, hid_hbm,
                      dwa, dwb, ring_ld, ring_id, lda, ldb, idsa, idsb,
                      rva, rvb, acc, swa, swb, sema, semb):
    wid = lax.axis_index("s") * 2 + lax.axis_index("c")
    lo = wid * NPW
    hi = lo + NPW
    hbase = wid * E2
    lane = lax.iota(jnp.int32, 16)
    neg_inf = jnp.full((16,), -jnp.inf, dtype=jnp.float32)

    def init(i, _):
        acc[pl.ds(i * 16, 16)] = neg_inf
        return ()

    lax.fori_loop(0, ACC_ROWS * WIDTH // 16, init, ())

    def flush_cond_maker(goal):
        def cond(carry):
            return carry[0] + goal <= carry[1]
        return cond

    def flush_body(carry):
        fl, p = carry
        flr = pl.multiple_of(fl % RING, FLUSH)
        flh = pl.multiple_of(hbase + fl, FLUSH)
        pltpu.sync_copy(ring_ld.at[pl.ds(flr, FLUSH)],
                        hld_hbm.at[pl.ds(flh, FLUSH)])
        pltpu.sync_copy(ring_id.at[pl.ds(flr, FLUSH)],
                        hid_hbm.at[pl.ds(flh, FLUSH)])
        return (fl + FLUSH, p)

    def fire_win(w, dw, sw):
        pltpu.async_copy(e1_hbm.at[pl.ds(w * W_SCAN, W_SCAN)], dw, sw)

    def scan_win(w, dw, sw, carry):
        p0, fl0 = carry
        pltpu.make_async_copy(e1_hbm.at[pl.ds(0, W_SCAN)], dw, sw).wait()

        def scan_grp(g, p):
            d = dw[pl.ds(g * 16, 16)]
            m = (d >= lo) & (d < hi)
            c = plsc.cumsum(jnp.where(m, 1, 0).astype(jnp.int32))
            pos = (p + c - 1) % RING
            plsc.store_scatter(ring_ld, [pos], d - lo, mask=m)
            ids = (w * W_SCAN + g * 16) + lane
            plsc.store_scatter(ring_id, [pos], ids, mask=m)
            return p + jnp.max(c)

        p1 = lax.fori_loop(0, W_SCAN // 16, scan_grp, p0)
        fl1, _ = lax.while_loop(flush_cond_maker(FLUSH), flush_body, (fl0, p1))
        return (p1, fl1)

    fire_win(0, dwa, swa)

    def win_pair(ww, carry):
        w = ww * 2
        fire_win(w + 1, dwb, swb)
        carry = scan_win(w, dwa, swa, carry)

        @pl.when(w + 2 < NWIN)
        def _():
            fire_win(w + 2, dwa, swa)

        carry = scan_win(w + 1, dwb, swb, carry)
        return carry

    p, fl = lax.fori_loop(0, NWIN // 2, win_pair, (0, 0))

    n_pad = (-(p - fl)) % FLUSH

    def pad_grp(t, _):
        pos = p + t * 16 + lane
        m = pos < p + n_pad
        plsc.store_scatter(ring_ld, [pos % RING],
                           jnp.full((16,), NPW, jnp.int32), mask=m)
        plsc.store_scatter(ring_id, [pos % RING],
                           jnp.zeros((16,), jnp.int32), mask=m)
        return ()

    lax.fori_loop(0, FLUSH // 16, pad_grp, ())
    fl, _ = lax.while_loop(flush_cond_maker(1), flush_body, (fl, p + n_pad))

    nb = fl // FLUSH

    def fire_batch(b, ld, idb, rv, sem):
        bofs = pl.multiple_of(hbase + b * FLUSH, FLUSH)
        pltpu.sync_copy(hld_hbm.at[pl.ds(bofs, FLUSH)], ld)
        pltpu.sync_copy(hid_hbm.at[pl.ds(bofs, FLUSH)], idb)
        pltpu.async_copy(emsg_hbm.at[idb], rv, sem)

    def proc_batch(ld, idb, rv, sem):
        pltpu.make_async_copy(emsg_hbm.at[idb], rv, sem).wait()

        def grp(g, _):
            lds16 = ld[pl.ds(g * 16, 16)]
            for j in range(16):
                bj = jnp.take(lds16, jnp.full((16,), j, jnp.int32))
                addr0 = bj * WIDTH + lane
                for f in range(8):
                    addr = addr0 + f * 16
                    av = plsc.load_gather(acc, [addr])
                    rv16 = rv[g * 16 + j, pl.ds(f * 16, 16)]
                    plsc.store_scatter(acc, [addr], jnp.maximum(av, rv16))
            return ()

        lax.fori_loop(0, FLUSH // 16, grp, ())

    @pl.when(nb > 0)
    def _():
        fire_batch(0, lda, idsa, rva, sema)

    def bpair(bb, _):
        b = bb * 2

        @pl.when(b + 1 < nb)
        def _():
            fire_batch(b + 1, ldb, idsb, rvb, semb)

        proc_batch(lda, idsa, rva, sema)

        @pl.when(b + 2 < nb)
        def _():
            fire_batch(b + 2, lda, idsa, rva, sema)

        @pl.when(b + 1 < nb)
        def _():
            proc_batch(ldb, idsb, rvb, semb)

        return ()

    lax.fori_loop(0, (nb + 1) // 2, bpair, ())

    pltpu.sync_copy(acc.at[pl.ds(0, NPW * WIDTH)],
                    maxes_hbm.at[pl.ds(pl.multiple_of(wid * NPW * WIDTH, 64),
                                       NPW * WIDTH)])


def _sc_scatter_max(e1, e_msg):
    mesh = plsc.VectorSubcoreMesh(core_axis_name="c", subcore_axis_name="s")
    f = pl.kernel(
        _scatter_max_body,
        out_type=[
            jax.ShapeDtypeStruct((NWORKERS * NPW * WIDTH,), jnp.float32),
            jax.ShapeDtypeStruct((NWORKERS * E2,), jnp.int32),
            jax.ShapeDtypeStruct((NWORKERS * E2,), jnp.int32),
        ],
        mesh=mesh,
        compiler_params=pltpu.CompilerParams(needs_layout_passes=False),
        scratch_types=(
            [pltpu.VMEM((W_SCAN,), jnp.int32)] * 2
            + [pltpu.VMEM((RING,), jnp.int32)] * 2
            + [pltpu.VMEM((FLUSH,), jnp.int32)] * 4
            + [pltpu.VMEM((FLUSH, WIDTH), jnp.float32)] * 2
            + [pltpu.VMEM((ACC_ROWS * WIDTH,), jnp.float32)]
            + [pltpu.SemaphoreType.DMA] * 4
        ),
    )
    maxes_flat, _, _ = f(e1, e_msg)
    return maxes_flat.reshape(NWORKERS * NPW, WIDTH)[:N_NODES]


def _edge_mlp_body(src_ref, dst_ref, ef_ref, w1_ref, w2_ref, b_ref, out_ref):
    src = src_ref[...]
    dst = dst_ref[...]
    diffs = dst - src
    h = jnp.dot(diffs, w1_ref[...], preferred_element_type=jnp.float32)
    ef = ef_ref[...]
    h = h + ef[:, 0:1] * w2_ref[0:1, :] + ef[:, 1:2] * w2_ref[1:2, :]
    h = jnp.maximum(h + b_ref[...], 0.0)
    out_ref[...] = diffs + h


def _edge_mlp(src_rows, dst_rows, ef2, W1, W2, b_edge):
    nblk = E2 // EDGE_BLK
    return pl.pallas_call(
        _edge_mlp_body,
        grid=(nblk,),
        in_specs=[
            pl.BlockSpec((EDGE_BLK, WIDTH), lambda i: (i, 0)),
            pl.BlockSpec((EDGE_BLK, WIDTH), lambda i: (i, 0)),
            pl.BlockSpec((EDGE_BLK, 2), lambda i: (i, 0)),
            pl.BlockSpec((WIDTH, WIDTH), lambda i: (0, 0)),
            pl.BlockSpec((2, WIDTH), lambda i: (0, 0)),
            pl.BlockSpec((1, WIDTH), lambda i: (0, 0)),
        ],
        out_specs=pl.BlockSpec((EDGE_BLK, WIDTH), lambda i: (i, 0)),
        out_shape=jax.ShapeDtypeStruct((E2, WIDTH), jnp.float32),
    )(src_rows, dst_rows, ef2, W1, W2, b_edge)


def _node_mlp_body(x_ref, mx_ref, wm1_ref, wm2_ref, b_ref, out_ref):
    x = x_ref[...]
    mx = mx_ref[...]
    mx = jnp.where(jnp.isneginf(mx), 0.0, mx)
    h = jnp.dot(x, wm1_ref[...], preferred_element_type=jnp.float32)
    h = h + jnp.dot(mx, wm2_ref[...], preferred_element_type=jnp.float32)
    h = jnp.maximum(h + b_ref[...], 0.0)
    out_ref[...] = x + h


def _node_mlp(x_p, maxes, Wm1, Wm2, b_mlp):
    nblk = N_NODES // NODE_BLK
    return pl.pallas_call(
        _node_mlp_body,
        grid=(nblk,),
        in_specs=[
            pl.BlockSpec((NODE_BLK, WIDTH), lambda i: (i, 0)),
            pl.BlockSpec((NODE_BLK, WIDTH), lambda i: (i, 0)),
            pl.BlockSpec((WIDTH, WIDTH), lambda i: (0, 0)),
            pl.BlockSpec((WIDTH, WIDTH), lambda i: (0, 0)),
            pl.BlockSpec((1, WIDTH), lambda i: (0, 0)),
        ],
        out_specs=pl.BlockSpec((NODE_BLK, WIDTH), lambda i: (i, 0)),
        out_shape=jax.ShapeDtypeStruct((N_NODES, WIDTH), jnp.float32),
    )(x_p, maxes, Wm1, Wm2, b_mlp)


def kernel(x_p, edge_indices, edge_feats, W_edge, b_edge, W_mlp, b_mlp):
    e0 = jnp.concatenate([edge_indices[0], edge_indices[1]])
    e1 = jnp.concatenate([edge_indices[1], edge_indices[0]])
    ef2 = jnp.concatenate([edge_feats, edge_feats], axis=0)
    W1 = W_edge[:WIDTH]
    W2 = W_edge[WIDTH:]
    Wm1 = W_mlp[:WIDTH]
    Wm2 = W_mlp[WIDTH:]

    src_rows, dst_rows = _sc_gather(x_p, e0, e1)

    e_msg = _edge_mlp(src_rows, dst_rows, ef2, W1, W2,
                      b_edge.reshape(1, WIDTH))

    maxes = _sc_scatter_max(e1, e_msg)

    return _node_mlp(x_p, maxes, Wm1, Wm2, b_mlp.reshape(1, WIDTH))


# speedup vs baseline: 3.0965x; 1.0567x over previous
import functools

import jax
import jax.numpy as jnp
from jax import lax
from jax.experimental import pallas as pl
from jax.experimental.pallas import tpu as pltpu
from jax.experimental.pallas import tpu_sc as plsc

N_NODES = 10000
WIDTH = 128
E2 = 640000

EDGE_BLK = 1024
NODE_BLK = 1000

NWORKERS = 32
GATHER_CHUNK = 200
EDGES_PER_W = E2 // NWORKERS


def _gather_body(x_hbm, e0_hbm, e1_hbm, src_hbm, dst_hbm,
                 i0a, i1a, i0b, i1b, sva, dva, svb, dvb,
                 s0a, s1a, s0b, s1b):
    wid = lax.axis_index("s") * 2 + lax.axis_index("c")
    wbase = wid * EDGES_PER_W

    def fire(c, i0, i1, sv, dv, s0, s1):
        base = wbase + c * GATHER_CHUNK
        pltpu.sync_copy(e0_hbm.at[pl.ds(base, GATHER_CHUNK)], i0)
        pltpu.sync_copy(e1_hbm.at[pl.ds(base, GATHER_CHUNK)], i1)
        pltpu.async_copy(x_hbm.at[i0], sv, s0)
        pltpu.async_copy(x_hbm.at[i1], dv, s1)

    def drain_wb(c, i0, i1, sv, dv, s0, s1):
        base = wbase + c * GATHER_CHUNK
        pltpu.make_async_copy(x_hbm.at[i0], sv, s0).wait()
        pltpu.make_async_copy(x_hbm.at[i1], dv, s1).wait()
        pltpu.sync_copy(sv, src_hbm.at[pl.ds(base, GATHER_CHUNK), :])
        pltpu.sync_copy(dv, dst_hbm.at[pl.ds(base, GATHER_CHUNK), :])

    fire(0, i0a, i1a, sva, dva, s0a, s1a)
    nch = EDGES_PER_W // GATHER_CHUNK

    def pair(cc, _):
        e = cc * 2
        fire(e + 1, i0b, i1b, svb, dvb, s0b, s1b)
        drain_wb(e, i0a, i1a, sva, dva, s0a, s1a)

        @pl.when(e + 2 < nch)
        def _():
            fire(e + 2, i0a, i1a, sva, dva, s0a, s1a)

        drain_wb(e + 1, i0b, i1b, svb, dvb, s0b, s1b)
        return ()

    lax.fori_loop(0, nch // 2, pair, ())


def _sc_gather(x_p, e0, e1):
    mesh = plsc.VectorSubcoreMesh(core_axis_name="c", subcore_axis_name="s")
    f = pl.kernel(
        _gather_body,
        out_type=[
            jax.ShapeDtypeStruct((E2, WIDTH), jnp.float32),
            jax.ShapeDtypeStruct((E2, WIDTH), jnp.float32),
        ],
        mesh=mesh,
        compiler_params=pltpu.CompilerParams(needs_layout_passes=False),
        scratch_types=(
            [pltpu.VMEM((GATHER_CHUNK,), jnp.int32)] * 4
            + [pltpu.VMEM((GATHER_CHUNK, WIDTH), jnp.float32)] * 4
            + [pltpu.SemaphoreType.DMA] * 4
        ),
    )
    return f(x_p, e0, e1)


NPW = 313
ACC_ROWS = NPW + 1
W_SCAN = 2560
RING = 4096
FLUSH = 256
NWIN = E2 // W_SCAN


def _scatter_max_body(e1_hbm, emsg_hbm, maxes_hbm, hpk_hbm,
                      dwa, dwb, ring, lda, ldb, idsa, idsb,
                      rva, rvb, acc, swa, swb, sema, semb):
    wid = lax.axis_index("s") * 2 + lax.axis_index("c")
    lo = wid * NPW
    hi = lo + NPW
    hbase = wid * E2
    lane = lax.iota(jnp.int32, 16)
    neg_inf = jnp.full((16,), -jnp.inf, dtype=jnp.float32)

    def init(i, _):
        acc[pl.ds(i * 16, 16)] = neg_inf
        return ()

    lax.fori_loop(0, ACC_ROWS * WIDTH // 16, init, ())

    def flush_cond_maker(goal, p_s):
        def cond(fl):
            return fl + goal <= p_s
        return cond

    def flush_body(fl):
        flr = pl.multiple_of(fl % RING, FLUSH)
        flh = pl.multiple_of(hbase + fl, FLUSH)
        pltpu.sync_copy(ring.at[pl.ds(flr, FLUSH)],
                        hpk_hbm.at[pl.ds(flh, FLUSH)])
        return fl + FLUSH

    def fire_win(w, dw, sw):
        pltpu.async_copy(e1_hbm.at[pl.ds(w * W_SCAN, W_SCAN)], dw, sw)

    def scan_win(w, dw, sw, carry):
        pv0, fl0 = carry
        pltpu.make_async_copy(e1_hbm.at[pl.ds(0, W_SCAN)], dw, sw).wait()

        def scan_grp(g, pv):
            d = dw[pl.ds(g * 16, 16)]
            m = (d >= lo) & (d < hi)
            c = plsc.cumsum(jnp.where(m, 1, 0).astype(jnp.int32))
            pos = (pv + c - 1) % RING
            val = lax.shift_left(d - lo, 20) + (w * W_SCAN + g * 16) + lane
            plsc.store_scatter(ring, [pos], val, mask=m)
            return pv + plsc.all_reduce_population_count(m)

        pv1 = lax.fori_loop(0, W_SCAN // 16, scan_grp, pv0)
        p_s = jnp.max(pv1)
        fl1 = lax.while_loop(flush_cond_maker(FLUSH, p_s), flush_body, fl0)
        return (pv1, fl1)

    fire_win(0, dwa, swa)

    def win_pair(ww, carry):
        w = ww * 2
        fire_win(w + 1, dwb, swb)
        carry = scan_win(w, dwa, swa, carry)

        @pl.when(w + 2 < NWIN)
        def _():
            fire_win(w + 2, dwa, swa)

        carry = scan_win(w + 1, dwb, swb, carry)
        return carry

    pv, fl = lax.fori_loop(0, NWIN // 2, win_pair,
                           (jnp.zeros((16,), jnp.int32), 0))
    p = jnp.max(pv)

    n_pad = (-(p - fl)) % FLUSH
    dummy = jnp.full((16,), NPW << 20, jnp.int32)

    def pad_grp(t, _):
        pos = p + t * 16 + lane
        m = pos < p + n_pad
        plsc.store_scatter(ring, [pos % RING], dummy, mask=m)
        return ()

    lax.fori_loop(0, FLUSH // 16, pad_grp, ())
    fl = lax.while_loop(flush_cond_maker(1, p + n_pad), flush_body, fl)

    nb = fl // FLUSH

    def fire_batch(b, ld, idb, rv, sem):
        bofs = pl.multiple_of(hbase + b * FLUSH, FLUSH)
        pltpu.sync_copy(hpk_hbm.at[pl.ds(bofs, FLUSH)], ld)

        def unpack(g, _):
            idb[pl.ds(g * 16, 16)] = ld[pl.ds(g * 16, 16)] & 0xFFFFF
            return ()

        lax.fori_loop(0, FLUSH // 16, unpack, ())
        pltpu.async_copy(emsg_hbm.at[idb], rv, sem)

    def proc_batch(ld, idb, rv, sem):
        pltpu.make_async_copy(emsg_hbm.at[idb], rv, sem).wait()

        def grp(g, _):
            lds16 = lax.shift_right_logical(ld[pl.ds(g * 16, 16)], 20)
            for j in range(16):
                bj = jnp.take(lds16, jnp.full((16,), j, jnp.int32))
                addr0 = bj * WIDTH + lane
                for f in range(8):
                    addr = addr0 + f * 16
                    av = plsc.load_gather(acc, [addr])
                    rv16 = rv[g * 16 + j, pl.ds(f * 16, 16)]
                    plsc.store_scatter(acc, [addr], jnp.maximum(av, rv16))
            return ()

        lax.fori_loop(0, FLUSH // 16, grp, ())

    @pl.when(nb > 0)
    def _():
        fire_batch(0, lda, idsa, rva, sema)

    def bpair(bb, _):
        b = bb * 2

        @pl.when(b + 1 < nb)
        def _():
            fire_batch(b + 1, ldb, idsb, rvb, semb)

        proc_batch(lda, idsa, rva, sema)

        @pl.when(b + 2 < nb)
        def _():
            fire_batch(b + 2, lda, idsa, rva, sema)

        @pl.when(b + 1 < nb)
        def _():
            proc_batch(ldb, idsb, rvb, semb)

        return ()

    lax.fori_loop(0, (nb + 1) // 2, bpair, ())

    pltpu.sync_copy(acc.at[pl.ds(0, NPW * WIDTH)],
                    maxes_hbm.at[pl.ds(pl.multiple_of(wid * NPW * WIDTH, 64),
                                       NPW * WIDTH)])


def _sc_scatter_max(e1, e_msg):
    mesh = plsc.VectorSubcoreMesh(core_axis_name="c", subcore_axis_name="s")
    f = pl.kernel(
        _scatter_max_body,
        out_type=[
            jax.ShapeDtypeStruct((NWORKERS * NPW * WIDTH,), jnp.float32),
            jax.ShapeDtypeStruct((NWORKERS * E2,), jnp.int32),
        ],
        mesh=mesh,
        compiler_params=pltpu.CompilerParams(needs_layout_passes=False),
        scratch_types=(
            [pltpu.VMEM((W_SCAN,), jnp.int32)] * 2
            + [pltpu.VMEM((RING,), jnp.int32)]
            + [pltpu.VMEM((FLUSH,), jnp.int32)] * 4
            + [pltpu.VMEM((FLUSH, WIDTH), jnp.float32)] * 2
            + [pltpu.VMEM((ACC_ROWS * WIDTH,), jnp.float32)]
            + [pltpu.SemaphoreType.DMA] * 4
        ),
    )
    maxes_flat, _ = f(e1, e_msg)
    return maxes_flat.reshape(NWORKERS * NPW, WIDTH)[:N_NODES]


def _edge_mlp_body(src_ref, dst_ref, ef_ref, w1_ref, w2_ref, b_ref, out_ref):
    src = src_ref[...]
    dst = dst_ref[...]
    diffs = dst - src
    h = jnp.dot(diffs, w1_ref[...], preferred_element_type=jnp.float32)
    ef = ef_ref[...]
    h = h + ef[:, 0:1] * w2_ref[0:1, :] + ef[:, 1:2] * w2_ref[1:2, :]
    h = jnp.maximum(h + b_ref[...], 0.0)
    out_ref[...] = diffs + h


def _edge_mlp(src_rows, dst_rows, ef2, W1, W2, b_edge):
    nblk = E2 // EDGE_BLK
    return pl.pallas_call(
        _edge_mlp_body,
        grid=(nblk,),
        in_specs=[
            pl.BlockSpec((EDGE_BLK, WIDTH), lambda i: (i, 0)),
            pl.BlockSpec((EDGE_BLK, WIDTH), lambda i: (i, 0)),
            pl.BlockSpec((EDGE_BLK, 2), lambda i: (i, 0)),
            pl.BlockSpec((WIDTH, WIDTH), lambda i: (0, 0)),
            pl.BlockSpec((2, WIDTH), lambda i: (0, 0)),
            pl.BlockSpec((1, WIDTH), lambda i: (0, 0)),
        ],
        out_specs=pl.BlockSpec((EDGE_BLK, WIDTH), lambda i: (i, 0)),
        out_shape=jax.ShapeDtypeStruct((E2, WIDTH), jnp.float32),
    )(src_rows, dst_rows, ef2, W1, W2, b_edge)


def _node_mlp_body(x_ref, mx_ref, wm1_ref, wm2_ref, b_ref, out_ref):
    x = x_ref[...]
    mx = mx_ref[...]
    mx = jnp.where(jnp.isneginf(mx), 0.0, mx)
    h = jnp.dot(x, wm1_ref[...], preferred_element_type=jnp.float32)
    h = h + jnp.dot(mx, wm2_ref[...], preferred_element_type=jnp.float32)
    h = jnp.maximum(h + b_ref[...], 0.0)
    out_ref[...] = x + h


def _node_mlp(x_p, maxes, Wm1, Wm2, b_mlp):
    nblk = N_NODES // NODE_BLK
    return pl.pallas_call(
        _node_mlp_body,
        grid=(nblk,),
        in_specs=[
            pl.BlockSpec((NODE_BLK, WIDTH), lambda i: (i, 0)),
            pl.BlockSpec((NODE_BLK, WIDTH), lambda i: (i, 0)),
            pl.BlockSpec((WIDTH, WIDTH), lambda i: (0, 0)),
            pl.BlockSpec((WIDTH, WIDTH), lambda i: (0, 0)),
            pl.BlockSpec((1, WIDTH), lambda i: (0, 0)),
        ],
        out_specs=pl.BlockSpec((NODE_BLK, WIDTH), lambda i: (i, 0)),
        out_shape=jax.ShapeDtypeStruct((N_NODES, WIDTH), jnp.float32),
    )(x_p, maxes, Wm1, Wm2, b_mlp)


def kernel(x_p, edge_indices, edge_feats, W_edge, b_edge, W_mlp, b_mlp):
    e0 = jnp.concatenate([edge_indices[0], edge_indices[1]])
    e1 = jnp.concatenate([edge_indices[1], edge_indices[0]])
    ef2 = jnp.concatenate([edge_feats, edge_feats], axis=0)
    W1 = W_edge[:WIDTH]
    W2 = W_edge[WIDTH:]
    Wm1 = W_mlp[:WIDTH]
    Wm2 = W_mlp[WIDTH:]

    src_rows, dst_rows = _sc_gather(x_p, e0, e1)

    e_msg = _edge_mlp(src_rows, dst_rows, ef2, W1, W2,
                      b_edge.reshape(1, WIDTH))

    maxes = _sc_scatter_max(e1, e_msg)

    return _node_mlp(x_p, maxes, Wm1, Wm2, b_mlp.reshape(1, WIDTH))


# speedup vs baseline: 5.4604x; 1.7634x over previous
import functools

import jax
import jax.numpy as jnp
from jax import lax
from jax.experimental import pallas as pl
from jax.experimental.pallas import tpu as pltpu
from jax.experimental.pallas import tpu_sc as plsc

N_NODES = 10000
WIDTH = 128
E2 = 640000

EDGE_BLK = 1024
NODE_BLK = 1000

NWORKERS = 32
GATHER_CHUNK = 160
EDGES_PER_W = E2 // NWORKERS


def _gather_body(x_hbm, e0_hbm, e1_hbm, src_hbm, dst_hbm, hpk_hbm, cnt_hbm,
                 i0a, i1a, i0b, i1b, sva, dva, svb, dvb, ring, ctv,
                 s0a, s1a, s0b, s1b):
    wid = lax.axis_index("s") * 2 + lax.axis_index("c")
    wbase = wid * EDGES_PER_W
    lo = wid * NPW
    hi = lo + NPW
    hbase = wid * E2
    lane = lax.iota(jnp.int32, 16)
    nch = EDGES_PER_W // GATHER_CHUNK

    def fire(c, i0, i1, sv, dv, s0, s1):
        base = wbase + c * GATHER_CHUNK
        pltpu.sync_copy(e0_hbm.at[pl.ds(base, GATHER_CHUNK)], i0)
        pltpu.sync_copy(e1_hbm.at[pl.ds(base, GATHER_CHUNK)], i1)
        pltpu.async_copy(x_hbm.at[i0], sv, s0)
        pltpu.async_copy(x_hbm.at[i1], dv, s1)

    def drain_wb(c, i0, i1, sv, dv, s0, s1):
        base = wbase + c * GATHER_CHUNK
        pltpu.make_async_copy(x_hbm.at[i0], sv, s0).wait()
        pltpu.make_async_copy(x_hbm.at[i1], dv, s1).wait()
        pltpu.sync_copy(sv, src_hbm.at[pl.ds(base, GATHER_CHUNK), :])
        pltpu.sync_copy(dv, dst_hbm.at[pl.ds(base, GATHER_CHUNK), :])

    def flush_cond_maker(goal, p_s):
        def cond(fl):
            return fl + goal <= p_s
        return cond

    def flush_body(fl):
        flr = pl.multiple_of(fl % RING, FLUSH)
        flh = pl.multiple_of(hbase + fl, FLUSH)
        pltpu.sync_copy(ring.at[pl.ds(flr, FLUSH)],
                        hpk_hbm.at[pl.ds(flh, FLUSH)])
        return fl + FLUSH

    def scan_chunk(c, i1, carry):
        pv0, fl0 = carry
        base = wbase + c * GATHER_CHUNK

        def scan_grp(g, pv):
            d = i1[pl.ds(g * 16, 16)]
            m = (d >= lo) & (d < hi)
            cc = plsc.cumsum(jnp.where(m, 1, 0).astype(jnp.int32))
            pos = (pv + cc - 1) % RING
            val = lax.shift_left(d - lo, 20) + (base + g * 16) + lane
            plsc.store_scatter(ring, [pos], val, mask=m)
            return pv + plsc.all_reduce_population_count(m)

        pv1 = lax.fori_loop(0, GATHER_CHUNK // 16, scan_grp, pv0)
        p_s = jnp.max(pv1)
        fl1 = lax.while_loop(flush_cond_maker(FLUSH, p_s), flush_body, fl0)
        return (pv1, fl1)

    fire(0, i0a, i1a, sva, dva, s0a, s1a)

    def pair(cc, carry):
        e = cc * 2

        @pl.when(e + 1 < nch)
        def _():
            fire(e + 1, i0b, i1b, svb, dvb, s0b, s1b)

        carry = scan_chunk(e, i1a, carry)
        drain_wb(e, i0a, i1a, sva, dva, s0a, s1a)

        @pl.when(e + 2 < nch)
        def _():
            fire(e + 2, i0a, i1a, sva, dva, s0a, s1a)

        def odd(carry):
            carry = scan_chunk(e + 1, i1b, carry)
            drain_wb(e + 1, i0b, i1b, svb, dvb, s0b, s1b)
            return carry

        carry = lax.cond(e + 1 < nch, odd, lambda car: car, carry)
        return carry

    pv, fl = lax.fori_loop(0, (nch + 1) // 2, pair,
                           (jnp.zeros((16,), jnp.int32), 0))
    p = jnp.max(pv)

    n_pad = (-(p - fl)) % FLUSH
    dummy = jnp.full((16,), NPW << 20, jnp.int32)

    def pad_grp(t, _):
        pos = p + t * 16 + lane
        m = pos < p + n_pad
        plsc.store_scatter(ring, [pos % RING], dummy, mask=m)
        return ()

    lax.fori_loop(0, FLUSH // 16, pad_grp, ())
    fl = lax.while_loop(flush_cond_maker(1, p + n_pad), flush_body, fl)

    ctv[pl.ds(0, 16)] = jnp.full((16,), fl, jnp.int32)
    pltpu.sync_copy(ctv, cnt_hbm.at[pl.ds(pl.multiple_of(wid * 16, 16), 16)])


def _sc_gather(x_p, e0, e1):
    mesh = plsc.VectorSubcoreMesh(core_axis_name="c", subcore_axis_name="s")
    f = pl.kernel(
        _gather_body,
        out_type=[
            jax.ShapeDtypeStruct((E2, WIDTH), jnp.float32),
            jax.ShapeDtypeStruct((E2, WIDTH), jnp.float32),
            jax.ShapeDtypeStruct((NWORKERS * E2,), jnp.int32),
            jax.ShapeDtypeStruct((NWORKERS * 16,), jnp.int32),
        ],
        mesh=mesh,
        compiler_params=pltpu.CompilerParams(needs_layout_passes=False),
        scratch_types=(
            [pltpu.VMEM((GATHER_CHUNK,), jnp.int32)] * 4
            + [pltpu.VMEM((GATHER_CHUNK, WIDTH), jnp.float32)] * 4
            + [pltpu.VMEM((RING,), jnp.int32)]
            + [pltpu.VMEM((16,), jnp.int32)]
            + [pltpu.SemaphoreType.DMA] * 4
        ),
    )
    return f(x_p, e0, e1)


NPW = 313
ACC_ROWS = NPW + 1
RING = 1024
FLUSH = 256


def _scatter_max_body(emsg_hbm, hpk_hbm, cnt_hbm, maxes_hbm,
                      lda, ldb, idsa, idsb, rva, rvb, acc, ctb, sema, semb):
    wid = lax.axis_index("s") * 2 + lax.axis_index("c")
    hbase = wid * E2
    lane = lax.iota(jnp.int32, 16)
    neg_inf = jnp.full((16,), -jnp.inf, dtype=jnp.float32)

    pltpu.sync_copy(cnt_hbm.at[pl.ds(pl.multiple_of(wid * 16, 16), 16)], ctb)

    def init(i, _):
        acc[pl.ds(i * 16, 16)] = neg_inf
        return ()

    lax.fori_loop(0, ACC_ROWS * WIDTH // 16, init, ())

    nb = jnp.max(ctb[pl.ds(0, 16)]) // FLUSH

    def fire_batch(b, ld, idb, rv, sem):
        bofs = pl.multiple_of(hbase + b * FLUSH, FLUSH)
        pltpu.sync_copy(hpk_hbm.at[pl.ds(bofs, FLUSH)], ld)

        def unpack(g, _):
            idb[pl.ds(g * 16, 16)] = ld[pl.ds(g * 16, 16)] & 0xFFFFF
            return ()

        lax.fori_loop(0, FLUSH // 16, unpack, ())
        pltpu.async_copy(emsg_hbm.at[idb], rv, sem)

    def proc_batch(ld, idb, rv, sem):
        pltpu.make_async_copy(emsg_hbm.at[idb], rv, sem).wait()

        def grp(g, _):
            lds16 = lax.shift_right_logical(ld[pl.ds(g * 16, 16)], 20)
            for j in range(16):
                bj = jnp.take(lds16, jnp.full((16,), j, jnp.int32))
                addr0 = bj * WIDTH + lane
                for f in range(8):
                    addr = addr0 + f * 16
                    av = plsc.load_gather(acc, [addr])
                    rv16 = rv[g * 16 + j, pl.ds(f * 16, 16)]
                    plsc.store_scatter(acc, [addr], jnp.maximum(av, rv16))
            return ()

        lax.fori_loop(0, FLUSH // 16, grp, ())

    @pl.when(nb > 0)
    def _():
        fire_batch(0, lda, idsa, rva, sema)

    def bpair(bb, _):
        b = bb * 2

        @pl.when(b + 1 < nb)
        def _():
            fire_batch(b + 1, ldb, idsb, rvb, semb)

        proc_batch(lda, idsa, rva, sema)

        @pl.when(b + 2 < nb)
        def _():
            fire_batch(b + 2, lda, idsa, rva, sema)

        @pl.when(b + 1 < nb)
        def _():
            proc_batch(ldb, idsb, rvb, semb)

        return ()

    lax.fori_loop(0, (nb + 1) // 2, bpair, ())

    pltpu.sync_copy(acc.at[pl.ds(0, NPW * WIDTH)],
                    maxes_hbm.at[pl.ds(pl.multiple_of(wid * NPW * WIDTH, 64),
                                       NPW * WIDTH)])


def _sc_scatter_max(e_msg, hpk, cnt):
    mesh = plsc.VectorSubcoreMesh(core_axis_name="c", subcore_axis_name="s")
    f = pl.kernel(
        _scatter_max_body,
        out_type=[
            jax.ShapeDtypeStruct((NWORKERS * NPW * WIDTH,), jnp.float32),
        ],
        mesh=mesh,
        compiler_params=pltpu.CompilerParams(needs_layout_passes=False),
        scratch_types=(
            [pltpu.VMEM((FLUSH,), jnp.int32)] * 4
            + [pltpu.VMEM((FLUSH, WIDTH), jnp.float32)] * 2
            + [pltpu.VMEM((ACC_ROWS * WIDTH,), jnp.float32)]
            + [pltpu.VMEM((16,), jnp.int32)]
            + [pltpu.SemaphoreType.DMA] * 2
        ),
    )
    (maxes_flat,) = f(e_msg, hpk, cnt)
    return maxes_flat.reshape(NWORKERS * NPW, WIDTH)[:N_NODES]


def _edge_mlp_body(src_ref, dst_ref, ef_ref, w1_ref, w2_ref, b_ref, out_ref):
    src = src_ref[...]
    dst = dst_ref[...]
    diffs = dst - src
    h = jnp.dot(diffs, w1_ref[...], preferred_element_type=jnp.float32)
    ef = ef_ref[...]
    h = h + ef[:, 0:1] * w2_ref[0:1, :] + ef[:, 1:2] * w2_ref[1:2, :]
    h = jnp.maximum(h + b_ref[...], 0.0)
    out_ref[...] = diffs + h


def _edge_mlp(src_rows, dst_rows, ef2, W1, W2, b_edge):
    nblk = E2 // EDGE_BLK
    return pl.pallas_call(
        _edge_mlp_body,
        grid=(nblk,),
        in_specs=[
            pl.BlockSpec((EDGE_BLK, WIDTH), lambda i: (i, 0)),
            pl.BlockSpec((EDGE_BLK, WIDTH), lambda i: (i, 0)),
            pl.BlockSpec((EDGE_BLK, 2), lambda i: (i, 0)),
            pl.BlockSpec((WIDTH, WIDTH), lambda i: (0, 0)),
            pl.BlockSpec((2, WIDTH), lambda i: (0, 0)),
            pl.BlockSpec((1, WIDTH), lambda i: (0, 0)),
        ],
        out_specs=pl.BlockSpec((EDGE_BLK, WIDTH), lambda i: (i, 0)),
        out_shape=jax.ShapeDtypeStruct((E2, WIDTH), jnp.float32),
    )(src_rows, dst_rows, ef2, W1, W2, b_edge)


def _node_mlp_body(x_ref, mx_ref, wm1_ref, wm2_ref, b_ref, out_ref):
    x = x_ref[...]
    mx = mx_ref[...]
    mx = jnp.where(jnp.isneginf(mx), 0.0, mx)
    h = jnp.dot(x, wm1_ref[...], preferred_element_type=jnp.float32)
    h = h + jnp.dot(mx, wm2_ref[...], preferred_element_type=jnp.float32)
    h = jnp.maximum(h + b_ref[...], 0.0)
    out_ref[...] = x + h


def _node_mlp(x_p, maxes, Wm1, Wm2, b_mlp):
    nblk = N_NODES // NODE_BLK
    return pl.pallas_call(
        _node_mlp_body,
        grid=(nblk,),
        in_specs=[
            pl.BlockSpec((NODE_BLK, WIDTH), lambda i: (i, 0)),
            pl.BlockSpec((NODE_BLK, WIDTH), lambda i: (i, 0)),
            pl.BlockSpec((WIDTH, WIDTH), lambda i: (0, 0)),
            pl.BlockSpec((WIDTH, WIDTH), lambda i: (0, 0)),
            pl.BlockSpec((1, WIDTH), lambda i: (0, 0)),
        ],
        out_specs=pl.BlockSpec((NODE_BLK, WIDTH), lambda i: (i, 0)),
        out_shape=jax.ShapeDtypeStruct((N_NODES, WIDTH), jnp.float32),
    )(x_p, maxes, Wm1, Wm2, b_mlp)


def kernel(x_p, edge_indices, edge_feats, W_edge, b_edge, W_mlp, b_mlp):
    e0 = jnp.concatenate([edge_indices[0], edge_indices[1]])
    e1 = jnp.concatenate([edge_indices[1], edge_indices[0]])
    ef2 = jnp.concatenate([edge_feats, edge_feats], axis=0)
    W1 = W_edge[:WIDTH]
    W2 = W_edge[WIDTH:]
    Wm1 = W_mlp[:WIDTH]
    Wm2 = W_mlp[WIDTH:]

    src_rows, dst_rows, hpk, cnt = _sc_gather(x_p, e0, e1)

    e_msg = _edge_mlp(src_rows, dst_rows, ef2, W1, W2,
                      b_edge.reshape(1, WIDTH))

    maxes = _sc_scatter_max(e_msg, hpk, cnt)

    return _node_mlp(x_p, maxes, Wm1, Wm2, b_mlp.reshape(1, WIDTH))
